# baseline (speedup 1.0000x reference)
import functools

import jax
import jax.numpy as jnp
from jax import lax
from jax.experimental import pallas as pl
from jax.experimental.pallas import tpu as pltpu
from jax.experimental.pallas import tpu_sc as plsc

_B = 16384
_D = 64
_L = 16
_NC = 2
_NS = 16
_NW = _NC * _NS
_BPW = _B // _NW
_C = 128
_NCH = _BPW // _C
_G = _C // _L
_MARGIN = 1.0


def _transe_body(ph, pt, pr, nh, nt, nr, ent, rel, out,
                 ph_i, pt_i, pr_i, nh_i, nt_i, nr_i,
                 ph_r, pt_r, pr_r, nh_r, nt_r, nr_r,
                 obuf, sem):
    wid = lax.axis_index("s") * _NC + lax.axis_index("c")
    base = wid * _BPW

    idx_pairs = ((ph, ph_i), (pt, pt_i), (pr, pr_i),
                 (nh, nh_i), (nt, nt_i), (nr, nr_i))
    cps = [pltpu.async_copy(src.at[pl.ds(base, _BPW)], dst, sem)
           for src, dst in idx_pairs]
    for cp in cps:
        cp.wait()

    iota = lax.iota(jnp.int32, _L)
    acc = jnp.zeros((_L,), jnp.float32)
    for k in range(_NCH):
        sl = pl.ds(k * _C, _C)
        gathers = [
            pltpu.async_copy(ent.at[ph_i.at[sl]], ph_r, sem),
            pltpu.async_copy(ent.at[pt_i.at[sl]], pt_r, sem),
            pltpu.async_copy(rel.at[pr_i.at[sl]], pr_r, sem),
            pltpu.async_copy(ent.at[nh_i.at[sl]], nh_r, sem),
            pltpu.async_copy(ent.at[nt_i.at[sl]], nt_r, sem),
            pltpu.async_copy(rel.at[nr_i.at[sl]], nr_r, sem),
        ]
        for cp in gathers:
            cp.wait()

        def gbody(g, acc):
            rows = g * _L + iota

            def cbody(c, rowsum):
                col = jnp.full((_L,), c, jnp.int32)
                a = plsc.load_gather(ph_r, [rows, col])
                b = plsc.load_gather(pr_r, [rows, col])
                t = plsc.load_gather(pt_r, [rows, col])
                d = plsc.load_gather(nh_r, [rows, col])
                e = plsc.load_gather(nr_r, [rows, col])
                f = plsc.load_gather(nt_r, [rows, col])
                return rowsum + (jnp.abs(a + b - t) - jnp.abs(d + e - f))

            rowsum = lax.fori_loop(0, _D, cbody, jnp.zeros((_L,), jnp.float32))
            return acc + jnp.maximum(rowsum + _MARGIN, 0.0)

        acc = lax.fori_loop(0, _G, gbody, acc)

    obuf[...] = acc
    pltpu.sync_copy(obuf, out.at[wid])


def _transe_partials(ph, pt, pr, nh, nt, nr, ent_emb, rel_emb):
    f32 = jnp.float32
    run = pl.kernel(
        _transe_body,
        mesh=plsc.VectorSubcoreMesh(core_axis_name="c", subcore_axis_name="s"),
        compiler_params=pltpu.CompilerParams(
            needs_layout_passes=False, use_tc_tiling_on_sc=False),
        out_type=jax.ShapeDtypeStruct((_NW, _L), f32),
        scratch_types=[
            pltpu.VMEM((_BPW,), jnp.int32),
            pltpu.VMEM((_BPW,), jnp.int32),
            pltpu.VMEM((_BPW,), jnp.int32),
            pltpu.VMEM((_BPW,), jnp.int32),
            pltpu.VMEM((_BPW,), jnp.int32),
            pltpu.VMEM((_BPW,), jnp.int32),
            pltpu.VMEM((_C, _D), f32),
            pltpu.VMEM((_C, _D), f32),
            pltpu.VMEM((_C, _D), f32),
            pltpu.VMEM((_C, _D), f32),
            pltpu.VMEM((_C, _D), f32),
            pltpu.VMEM((_C, _D), f32),
            pltpu.VMEM((_L,), f32),
            pltpu.SemaphoreType.DMA,
        ],
    )
    return run(ph, pt, pr, nh, nt, nr, ent_emb, rel_emb)


def kernel(x, ent_emb, rel_emb):
    ph, pt, pr = x[:, 0], x[:, 1], x[:, 2]
    nh, nt, nr = x[:, 3], x[:, 4], x[:, 5]
    partials = _transe_partials(ph, pt, pr, nh, nt, nr, ent_emb, rel_emb)
    return jnp.sum(partials)


# speedup vs baseline: 1.0086x; 1.0086x over previous
import jax
import jax.numpy as jnp
from jax import lax
from jax.experimental import pallas as pl
from jax.experimental.pallas import tpu as pltpu
from jax.experimental.pallas import tpu_sc as plsc

_B = 16384
_D = 64
_L = 16
_NC = 2
_NS = 16
_NW = _NC * _NS
_BPW = _B // _NW
_C = 128
_NCH = _BPW // _C
_G = _C // _L
_MARGIN = 1.0
_UNROLL = 4


def _transe_body(x, ent, rel, out, xbuf, idx, rows0, rows1, obuf,
                 sem_x, sem0, sem1):
    wid = lax.axis_index("s") * _NC + lax.axis_index("c")
    base = wid * _BPW

    pltpu.async_copy(x.at[pl.ds(base, _BPW)], xbuf, sem_x).wait()

    iota = lax.iota(jnp.int32, _L)

    def tbody(g, _):
        rows = g * _L + iota
        for j in range(6):
            col = jnp.full((_L,), j, jnp.int32)
            idx[j, pl.ds(g * _L, _L)] = plsc.load_gather(xbuf, [rows, col])
        return 0

    lax.fori_loop(0, _BPW // _L, tbody, 0)

    rowbufs = (rows0, rows1)
    sems = (sem0, sem1)

    def fire(k, s):
        sl = pl.ds(k * _C, _C)
        rb, sm = rowbufs[s], sems[s]
        return [
            pltpu.async_copy(ent.at[idx.at[0, sl]], rb.at[0], sm),
            pltpu.async_copy(ent.at[idx.at[1, sl]], rb.at[1], sm),
            pltpu.async_copy(rel.at[idx.at[2, sl]], rb.at[2], sm),
            pltpu.async_copy(ent.at[idx.at[3, sl]], rb.at[3], sm),
            pltpu.async_copy(ent.at[idx.at[4, sl]], rb.at[4], sm),
            pltpu.async_copy(rel.at[idx.at[5, sl]], rb.at[5], sm),
        ]

    acc = jnp.zeros((_L,), jnp.float32)
    cps = fire(0, 0)
    for k in range(_NCH):
        nxt = fire(k + 1, (k + 1) % 2) if k + 1 < _NCH else None
        for cp in cps:
            cp.wait()
        rb = rowbufs[k % 2]

        def gbody(g, acc, rb=rb):
            rows = g * _L + iota

            def cbody(ci, rowsum):
                for u in range(_UNROLL):
                    c = ci * _UNROLL + u
                    col = jnp.full((_L,), c, jnp.int32)
                    a = plsc.load_gather(rb.at[0], [rows, col])
                    t = plsc.load_gather(rb.at[1], [rows, col])
                    b = plsc.load_gather(rb.at[2], [rows, col])
                    d = plsc.load_gather(rb.at[3], [rows, col])
                    e = plsc.load_gather(rb.at[4], [rows, col])
                    f = plsc.load_gather(rb.at[5], [rows, col])
                    rowsum = rowsum + (jnp.abs(a + b - t) - jnp.abs(d + f - e))
                return rowsum

            rowsum = lax.fori_loop(0, _D // _UNROLL, cbody,
                                   jnp.zeros((_L,), jnp.float32))
            return acc + jnp.maximum(rowsum + _MARGIN, 0.0)

        acc = lax.fori_loop(0, _G, gbody, acc)
        cps = nxt

    obuf[...] = acc
    pltpu.sync_copy(obuf, out.at[wid])


def _transe_partials(x, ent_emb, rel_emb):
    f32 = jnp.float32
    run = pl.kernel(
        _transe_body,
        mesh=plsc.VectorSubcoreMesh(core_axis_name="c", subcore_axis_name="s"),
        compiler_params=pltpu.CompilerParams(
            needs_layout_passes=False, use_tc_tiling_on_sc=False),
        out_type=jax.ShapeDtypeStruct((_NW, _L), f32),
        scratch_types=[
            pltpu.VMEM((_BPW, 6), jnp.int32),
            pltpu.VMEM((6, _BPW), jnp.int32),
            pltpu.VMEM((6, _C, _D), f32),
            pltpu.VMEM((6, _C, _D), f32),
            pltpu.VMEM((_L,), f32),
            pltpu.SemaphoreType.DMA,
            pltpu.SemaphoreType.DMA,
            pltpu.SemaphoreType.DMA,
        ],
    )
    return run(x, ent_emb, rel_emb)


def kernel(x, ent_emb, rel_emb):
    return jnp.sum(_transe_partials(x, ent_emb, rel_emb))
